# Initial kernel scaffold; baseline (speedup 1.0000x reference)
import jax
import jax.numpy as jnp
from jax.experimental import pallas as pl


def kernel(inputs, table):
    raise NotImplementedError("write your pallas kernel here")


# speedup vs baseline: 34.3972x; 34.3972x over previous
import functools

import jax
import jax.numpy as jnp
import numpy as np
from jax import lax
from jax.experimental import pallas as pl
from jax.experimental.pallas import tpu as pltpu
from jax.experimental.pallas import tpu_sc as plsc

_IN_CHANNELS = 3
_N_LEVELS = 16
_F_PER = 2
_LOG2_T = 19
_BASE = 16
_MAXR = 2048
_T = 1 << _LOG2_T
_PER_LEVEL_SCALE = (_MAXR / _BASE) ** (1.0 / _N_LEVELS)
_PRIME1 = 2654435761
_PRIME2 = 805459861
_N_POINTS = 524288

_NC, _NS, _L = 2, 16, 16
_NW = _NC * _NS
_P = 1024
_PTS_PER_W = _N_POINTS // _NW
_CHUNKS = _PTS_PER_W // _P
_GSZ = 16 * _P
_ISL = 128
_NG = _GSZ // _ISL


def _resolutions():
    return [int(np.floor(_BASE * _PER_LEVEL_SCALE ** l)) for l in range(_N_LEVELS)]


def _sc_body(x_hbm, y_hbm, z_hbm, tab_hbm, out_hbm,
             xs, ys, zs, idx_a, idx_b, gb_a, gb_b, outb, sem_a, sem_b):
    wid = lax.axis_index("s") * _NC + lax.axis_index("c")
    iota = lax.iota(jnp.int32, _L)
    dup = lax.shift_right_logical(iota, 1)
    col = lax.bitwise_and(iota, 1)
    resolutions = _resolutions()

    def compute_idx(l, res, idxbuf, s, _):
        rowd = s * 8 + dup
        xd = plsc.load_gather(xs, [rowd]) * jnp.float32(res)
        yd = plsc.load_gather(ys, [rowd]) * jnp.float32(res)
        zd = plsc.load_gather(zs, [rowd]) * jnp.float32(res)
        xi = xd.astype(jnp.int32)
        yi = yd.astype(jnp.int32)
        zi = zd.astype(jnp.int32)
        sl = s * 16
        if (res + 1) ** _IN_CHANNELS <= _T:
            s1 = res + 1
            s2 = s1 * s1
            b00 = yi * s1 + zi * s2 + (xi + l * _T)
            combos = (b00, b00 + s2, b00 + s1, b00 + s1 + s2)
            for c in range(8):
                i, jj, k = c >> 2, (c >> 1) & 1, c & 1
                e = lax.shift_left(combos[jj * 2 + k] + i, 1) + col
                idxbuf[pl.ds(c * 2 * _P + sl, _L)] = e
        else:
            xu = xi.astype(jnp.uint32)
            hy0 = yi.astype(jnp.uint32) * jnp.uint32(_PRIME1)
            hz0 = zi.astype(jnp.uint32) * jnp.uint32(_PRIME2)
            hx1 = xu + jnp.uint32(1)
            hy1 = hy0 + jnp.uint32(_PRIME1)
            hz1 = hz0 + jnp.uint32(_PRIME2)
            hyz = (hy0 ^ hz0, hy0 ^ hz1, hy1 ^ hz0, hy1 ^ hz1)
            mask = jnp.uint32(_T - 1)
            for c in range(8):
                i, jj, k = c >> 2, (c >> 1) & 1, c & 1
                h = (xu if i == 0 else hx1) ^ hyz[jj * 2 + k]
                e = lax.shift_left((h & mask).astype(jnp.int32) + l * _T, 1) + col
                idxbuf[pl.ds(c * 2 * _P + sl, _L)] = e
        return _

    def fire(idxbuf, gbuf, sem, j, _):
        pltpu.async_copy(
            tab_hbm.at[idxbuf.at[pl.ds(j * _ISL, _ISL)]],
            gbuf.at[pl.ds(j * _ISL, _ISL)], sem)
        return _

    def combine(l, res, gbuf, s, _):
        rowd = s * 8 + dup
        xd = plsc.load_gather(xs, [rowd]) * jnp.float32(res)
        yd = plsc.load_gather(ys, [rowd]) * jnp.float32(res)
        zd = plsc.load_gather(zs, [rowd]) * jnp.float32(res)
        fx = xd - xd.astype(jnp.int32).astype(jnp.float32)
        fy = yd - yd.astype(jnp.int32).astype(jnp.float32)
        fz = zd - zd.astype(jnp.int32).astype(jnp.float32)
        sl = s * 16
        v = [gbuf[pl.ds(c * 2 * _P + sl, _L)] for c in range(8)]
        m00 = v[0] + fz * (v[1] - v[0])
        m01 = v[2] + fz * (v[3] - v[2])
        m10 = v[4] + fz * (v[5] - v[4])
        m11 = v[6] + fz * (v[7] - v[6])
        n0 = m00 + fy * (m01 - m00)
        n1 = m10 + fy * (m11 - m10)
        r = n0 + fx * (n1 - n0)
        plsc.store_scatter(outb, [rowd, col + 2 * l], r)
        return _

    def chunk_body(ci, _):
        base = wid * _PTS_PER_W + ci * _P
        pltpu.sync_copy(x_hbm.at[pl.ds(base, _P)], xs)
        pltpu.sync_copy(y_hbm.at[pl.ds(base, _P)], ys)
        pltpu.sync_copy(z_hbm.at[pl.ds(base, _P)], zs)
        prev = None
        for l in range(_N_LEVELS):
            res = resolutions[l]
            idxbuf = idx_a if l % 2 == 0 else idx_b
            gbuf = gb_a if l % 2 == 0 else gb_b
            sem = sem_a if l % 2 == 0 else sem_b
            lax.fori_loop(0, _P // 8,
                          functools.partial(compute_idx, l, res, idxbuf), None)
            lax.fori_loop(0, _NG, functools.partial(fire, idxbuf, gbuf, sem),
                          None)
            if prev is not None:
                pl_, gb_, sem_ = prev
                pltpu.make_async_copy(tab_hbm.at[pl.ds(0, _GSZ)], gb_,
                                      sem_).wait()
                lax.fori_loop(0, _P // 8,
                              functools.partial(combine, pl_,
                                                resolutions[pl_], gb_), None)
            prev = (l, gbuf, sem)
        pl_, gb_, sem_ = prev
        pltpu.make_async_copy(tab_hbm.at[pl.ds(0, _GSZ)], gb_, sem_).wait()
        lax.fori_loop(0, _P // 8,
                      functools.partial(combine, pl_, resolutions[pl_], gb_),
                      None)
        pltpu.sync_copy(outb, out_hbm.at[pl.ds(base, _P)])
        return _

    lax.fori_loop(0, _CHUNKS, chunk_body, None)


@jax.jit
def _hash_grid(x, y, z, tab):
    mesh = plsc.VectorSubcoreMesh(core_axis_name="c", subcore_axis_name="s")
    f = pl.kernel(
        _sc_body,
        out_type=jax.ShapeDtypeStruct((_N_POINTS, _N_LEVELS * _F_PER), jnp.float32),
        mesh=mesh,
        compiler_params=pltpu.CompilerParams(
            needs_layout_passes=False, use_tc_tiling_on_sc=False),
        scratch_types=[
            pltpu.VMEM((_P,), jnp.float32),
            pltpu.VMEM((_P,), jnp.float32),
            pltpu.VMEM((_P,), jnp.float32),
            pltpu.VMEM((_GSZ,), jnp.int32),
            pltpu.VMEM((_GSZ,), jnp.int32),
            pltpu.VMEM((_GSZ,), jnp.float32),
            pltpu.VMEM((_GSZ,), jnp.float32),
            pltpu.VMEM((_P, _N_LEVELS * _F_PER), jnp.float32),
            pltpu.SemaphoreType.DMA,
            pltpu.SemaphoreType.DMA,
        ],
    )
    return f(x, y, z, tab)


def kernel(inputs, table):
    x = inputs[:, 0]
    y = inputs[:, 1]
    z = inputs[:, 2]
    tab = table.reshape(_N_LEVELS * _T * _F_PER)
    return _hash_grid(x, y, z, tab)


# speedup vs baseline: 34.6379x; 1.0070x over previous
import functools

import jax
import jax.numpy as jnp
import numpy as np
from jax import lax
from jax.experimental import pallas as pl
from jax.experimental.pallas import tpu as pltpu
from jax.experimental.pallas import tpu_sc as plsc

_IN_CHANNELS = 3
_N_LEVELS = 16
_F_PER = 2
_LOG2_T = 19
_BASE = 16
_MAXR = 2048
_T = 1 << _LOG2_T
_PER_LEVEL_SCALE = (_MAXR / _BASE) ** (1.0 / _N_LEVELS)
_PRIME1 = 2654435761
_PRIME2 = 805459861
_N_POINTS = 524288

_NC, _NS, _L = 2, 16, 16
_NW = _NC * _NS
_P = 1024
_PTS_PER_W = _N_POINTS // _NW
_CHUNKS = _PTS_PER_W // _P
_GSZ = 16 * _P
_ISL = 128
_NG = _GSZ // _ISL


def _resolutions():
    return [int(np.floor(_BASE * _PER_LEVEL_SCALE ** l)) for l in range(_N_LEVELS)]


def _sc_body(x_hbm, y_hbm, z_hbm, tab_hbm, out_hbm,
             xs, ys, zs, idx_a, idx_b, gb_a, gb_b, outb, sem_a, sem_b):
    wid = lax.axis_index("s") * _NC + lax.axis_index("c")
    iota = lax.iota(jnp.int32, _L)
    dup = lax.shift_right_logical(iota, 1)
    col = lax.bitwise_and(iota, 1)
    resolutions = _resolutions()

    def compute_idx(l, res, idxbuf, s, _):
        rowd = s * 8 + dup
        xd = plsc.load_gather(xs, [rowd]) * jnp.float32(res)
        yd = plsc.load_gather(ys, [rowd]) * jnp.float32(res)
        zd = plsc.load_gather(zs, [rowd]) * jnp.float32(res)
        xi = xd.astype(jnp.int32)
        yi = yd.astype(jnp.int32)
        zi = zd.astype(jnp.int32)
        sl = s * 16
        if (res + 1) ** _IN_CHANNELS <= _T:
            s1 = res + 1
            s2 = s1 * s1
            b00 = yi * s1 + zi * s2 + (xi + l * _T)
            combos = (b00, b00 + s2, b00 + s1, b00 + s1 + s2)
            for c in range(8):
                i, jj, k = c >> 2, (c >> 1) & 1, c & 1
                e = lax.shift_left(combos[jj * 2 + k] + i, 1) + col
                idxbuf[pl.ds(c * 2 * _P + sl, _L)] = e
        else:
            xu = xi.astype(jnp.uint32)
            hy0 = yi.astype(jnp.uint32) * jnp.uint32(_PRIME1)
            hz0 = zi.astype(jnp.uint32) * jnp.uint32(_PRIME2)
            hx1 = xu + jnp.uint32(1)
            hy1 = hy0 + jnp.uint32(_PRIME1)
            hz1 = hz0 + jnp.uint32(_PRIME2)
            hyz = (hy0 ^ hz0, hy0 ^ hz1, hy1 ^ hz0, hy1 ^ hz1)
            mask = jnp.uint32(_T - 1)
            for c in range(8):
                i, jj, k = c >> 2, (c >> 1) & 1, c & 1
                h = (xu if i == 0 else hx1) ^ hyz[jj * 2 + k]
                e = lax.shift_left((h & mask).astype(jnp.int32) + l * _T, 1) + col
                idxbuf[pl.ds(c * 2 * _P + sl, _L)] = e
        return _

    def fire(idxbuf, gbuf, sem, j, _):
        pltpu.async_copy(
            tab_hbm.at[idxbuf.at[pl.ds(j * _ISL, _ISL)]],
            gbuf.at[pl.ds(j * _ISL, _ISL)], sem)
        return _

    def fire_all(idxbuf, gbuf, sem):
        pltpu.async_copy(tab_hbm.at[idxbuf], gbuf, sem)

    def combine(l, res, gbuf, s, _):
        rowd = s * 8 + dup
        xd = plsc.load_gather(xs, [rowd]) * jnp.float32(res)
        yd = plsc.load_gather(ys, [rowd]) * jnp.float32(res)
        zd = plsc.load_gather(zs, [rowd]) * jnp.float32(res)
        fx = xd - xd.astype(jnp.int32).astype(jnp.float32)
        fy = yd - yd.astype(jnp.int32).astype(jnp.float32)
        fz = zd - zd.astype(jnp.int32).astype(jnp.float32)
        sl = s * 16
        v = [gbuf[pl.ds(c * 2 * _P + sl, _L)] for c in range(8)]
        m00 = v[0] + fz * (v[1] - v[0])
        m01 = v[2] + fz * (v[3] - v[2])
        m10 = v[4] + fz * (v[5] - v[4])
        m11 = v[6] + fz * (v[7] - v[6])
        n0 = m00 + fy * (m01 - m00)
        n1 = m10 + fy * (m11 - m10)
        r = n0 + fx * (n1 - n0)
        plsc.store_scatter(outb, [rowd * 32 + (col + 2 * l)], r)
        return _

    def chunk_body(ci, _):
        base = wid * _PTS_PER_W + ci * _P
        pltpu.sync_copy(x_hbm.at[pl.ds(base, _P)], xs)
        pltpu.sync_copy(y_hbm.at[pl.ds(base, _P)], ys)
        pltpu.sync_copy(z_hbm.at[pl.ds(base, _P)], zs)
        prev = None
        for l in range(_N_LEVELS):
            res = resolutions[l]
            idxbuf = idx_a if l % 2 == 0 else idx_b
            gbuf = gb_a if l % 2 == 0 else gb_b
            sem = sem_a if l % 2 == 0 else sem_b
            lax.fori_loop(0, _P // 8,
                          functools.partial(compute_idx, l, res, idxbuf), None)
            fire_all(idxbuf, gbuf, sem)
            if prev is not None:
                pl_, gb_, sem_ = prev
                pltpu.make_async_copy(tab_hbm.at[pl.ds(0, _GSZ)], gb_,
                                      sem_).wait()
                lax.fori_loop(0, _P // 8,
                              functools.partial(combine, pl_,
                                                resolutions[pl_], gb_), None)
            prev = (l, gbuf, sem)
        pl_, gb_, sem_ = prev
        pltpu.make_async_copy(tab_hbm.at[pl.ds(0, _GSZ)], gb_, sem_).wait()
        lax.fori_loop(0, _P // 8,
                      functools.partial(combine, pl_, resolutions[pl_], gb_),
                      None)
        pltpu.sync_copy(outb, out_hbm.at[pl.ds(base * 32, _P * 32)])
        return _

    lax.fori_loop(0, _CHUNKS, chunk_body, None)


@jax.jit
def _hash_grid(x, y, z, tab):
    mesh = plsc.VectorSubcoreMesh(core_axis_name="c", subcore_axis_name="s")
    f = pl.kernel(
        _sc_body,
        out_type=jax.ShapeDtypeStruct((_N_POINTS * _N_LEVELS * _F_PER,), jnp.float32),
        mesh=mesh,
        compiler_params=pltpu.CompilerParams(
            needs_layout_passes=False, use_tc_tiling_on_sc=False),
        scratch_types=[
            pltpu.VMEM((_P,), jnp.float32),
            pltpu.VMEM((_P,), jnp.float32),
            pltpu.VMEM((_P,), jnp.float32),
            pltpu.VMEM((_GSZ,), jnp.int32),
            pltpu.VMEM((_GSZ,), jnp.int32),
            pltpu.VMEM((_GSZ,), jnp.float32),
            pltpu.VMEM((_GSZ,), jnp.float32),
            pltpu.VMEM((_P * _N_LEVELS * _F_PER,), jnp.float32),
            pltpu.SemaphoreType.DMA,
            pltpu.SemaphoreType.DMA,
        ],
    )
    return f(x, y, z, tab)


def kernel(inputs, table):
    x = inputs[:, 0]
    y = inputs[:, 1]
    z = inputs[:, 2]
    tab = table.reshape(_N_LEVELS * _T * _F_PER)
    out = _hash_grid(x, y, z, tab)
    return out.reshape(_N_POINTS, _N_LEVELS * _F_PER)


# speedup vs baseline: 102.8256x; 2.9686x over previous
import functools

import jax
import jax.numpy as jnp
import numpy as np
from jax import lax
from jax.experimental import pallas as pl
from jax.experimental.pallas import tpu as pltpu
from jax.experimental.pallas import tpu_sc as plsc

_IN_CHANNELS = 3
_N_LEVELS = 16
_F_PER = 2
_LOG2_T = 19
_BASE = 16
_MAXR = 2048
_T = 1 << _LOG2_T
_PER_LEVEL_SCALE = (_MAXR / _BASE) ** (1.0 / _N_LEVELS)
_PRIME1 = 2654435761
_PRIME2 = 805459861
_N_POINTS = 524288

_NC, _NS, _L = 2, 16, 16
_NW = _NC * _NS
_P = 1024
_PTS_PER_W = _N_POINTS // _NW
_CHUNKS = _PTS_PER_W // _P
_GSZ = 16 * _P
_ISL = 128
_NG = _GSZ // _ISL


def _resolutions():
    return [int(np.floor(_BASE * _PER_LEVEL_SCALE ** l)) for l in range(_N_LEVELS)]


def _sc_body(x_hbm, y_hbm, z_hbm, tab_hbm, out_hbm,
             xs, ys, zs, idx_a, idx_b, gb_a, gb_b, outb, sem_a, sem_b):
    wid = lax.axis_index("s") * _NC + lax.axis_index("c")
    iota = lax.iota(jnp.int32, _L)
    dup = lax.shift_right_logical(iota, 1)
    col = lax.bitwise_and(iota, 1)
    resolutions = _resolutions()

    def compute_idx(l, res, idxbuf, s, _):
        rowd = s * 8 + dup
        xd = plsc.load_gather(xs, [rowd]) * jnp.float32(res)
        yd = plsc.load_gather(ys, [rowd]) * jnp.float32(res)
        zd = plsc.load_gather(zs, [rowd]) * jnp.float32(res)
        xi = xd.astype(jnp.int32)
        yi = yd.astype(jnp.int32)
        zi = zd.astype(jnp.int32)
        sl = s * 16
        flvl = col * 128 + l * _T * 2
        if (res + 1) ** _IN_CHANNELS <= _T:
            s1 = res + 1
            s2 = s1 * s1
            b00 = yi * s1 + zi * s2 + xi
            combos = (b00, b00 + s2, b00 + s1, b00 + s1 + s2)
            for c in range(8):
                i, jj, k = c >> 2, (c >> 1) & 1, c & 1
                t = combos[jj * 2 + k] + i
                e = (lax.shift_left(lax.shift_right_logical(t, 7), 8)
                     + lax.bitwise_and(t, 127) + flvl)
                idxbuf[pl.ds(c * 2 * _P + sl, _L)] = e
        else:
            xu = xi.astype(jnp.uint32)
            hy0 = yi.astype(jnp.uint32) * jnp.uint32(_PRIME1)
            hz0 = zi.astype(jnp.uint32) * jnp.uint32(_PRIME2)
            hx1 = xu + jnp.uint32(1)
            hy1 = hy0 + jnp.uint32(_PRIME1)
            hz1 = hz0 + jnp.uint32(_PRIME2)
            hyz = (hy0 ^ hz0, hy0 ^ hz1, hy1 ^ hz0, hy1 ^ hz1)
            mask = jnp.uint32(_T - 1)
            for c in range(8):
                i, jj, k = c >> 2, (c >> 1) & 1, c & 1
                h = (xu if i == 0 else hx1) ^ hyz[jj * 2 + k]
                t = (h & mask).astype(jnp.int32)
                e = (lax.shift_left(lax.shift_right_logical(t, 7), 8)
                     + lax.bitwise_and(t, 127) + flvl)
                idxbuf[pl.ds(c * 2 * _P + sl, _L)] = e
        return _

    def fire(idxbuf, gbuf, sem, j, _):
        pltpu.async_copy(
            tab_hbm.at[idxbuf.at[pl.ds(j * _ISL, _ISL)]],
            gbuf.at[pl.ds(j * _ISL, _ISL)], sem)
        return _

    def fire_all(idxbuf, gbuf, sem):
        pltpu.async_copy(tab_hbm.at[idxbuf], gbuf, sem)

    def combine(l, res, gbuf, s, _):
        rowd = s * 8 + dup
        xd = plsc.load_gather(xs, [rowd]) * jnp.float32(res)
        yd = plsc.load_gather(ys, [rowd]) * jnp.float32(res)
        zd = plsc.load_gather(zs, [rowd]) * jnp.float32(res)
        fx = xd - xd.astype(jnp.int32).astype(jnp.float32)
        fy = yd - yd.astype(jnp.int32).astype(jnp.float32)
        fz = zd - zd.astype(jnp.int32).astype(jnp.float32)
        sl = s * 16
        v = [gbuf[pl.ds(c * 2 * _P + sl, _L)] for c in range(8)]
        m00 = v[0] + fz * (v[1] - v[0])
        m01 = v[2] + fz * (v[3] - v[2])
        m10 = v[4] + fz * (v[5] - v[4])
        m11 = v[6] + fz * (v[7] - v[6])
        n0 = m00 + fy * (m01 - m00)
        n1 = m10 + fy * (m11 - m10)
        r = n0 + fx * (n1 - n0)
        plsc.store_scatter(outb, [rowd * 32 + (col + 2 * l)], r)
        return _

    def chunk_body(ci, _):
        base = wid * _PTS_PER_W + ci * _P
        pltpu.sync_copy(x_hbm.at[pl.ds(base, _P)], xs)
        pltpu.sync_copy(y_hbm.at[pl.ds(base, _P)], ys)
        pltpu.sync_copy(z_hbm.at[pl.ds(base, _P)], zs)
        prev = None
        for l in range(_N_LEVELS):
            res = resolutions[l]
            idxbuf = idx_a if l % 2 == 0 else idx_b
            gbuf = gb_a if l % 2 == 0 else gb_b
            sem = sem_a if l % 2 == 0 else sem_b
            lax.fori_loop(0, _P // 8,
                          functools.partial(compute_idx, l, res, idxbuf), None)
            fire_all(idxbuf, gbuf, sem)
            if prev is not None:
                pl_, gb_, sem_ = prev
                pltpu.make_async_copy(tab_hbm.at[pl.ds(0, _GSZ)], gb_,
                                      sem_).wait()
                lax.fori_loop(0, _P // 8,
                              functools.partial(combine, pl_,
                                                resolutions[pl_], gb_), None)
            prev = (l, gbuf, sem)
        pl_, gb_, sem_ = prev
        pltpu.make_async_copy(tab_hbm.at[pl.ds(0, _GSZ)], gb_, sem_).wait()
        lax.fori_loop(0, _P // 8,
                      functools.partial(combine, pl_, resolutions[pl_], gb_),
                      None)
        pltpu.sync_copy(outb, out_hbm.at[pl.ds(base * 32, _P * 32)])
        return _

    lax.fori_loop(0, _CHUNKS, chunk_body, None)


@jax.jit
def _hash_grid(x, y, z, tab):
    mesh = plsc.VectorSubcoreMesh(core_axis_name="c", subcore_axis_name="s")
    f = pl.kernel(
        _sc_body,
        out_type=jax.ShapeDtypeStruct((_N_POINTS * _N_LEVELS * _F_PER,), jnp.float32),
        mesh=mesh,
        compiler_params=pltpu.CompilerParams(
            needs_layout_passes=False, use_tc_tiling_on_sc=False),
        scratch_types=[
            pltpu.VMEM((_P,), jnp.float32),
            pltpu.VMEM((_P,), jnp.float32),
            pltpu.VMEM((_P,), jnp.float32),
            pltpu.VMEM((_GSZ,), jnp.int32),
            pltpu.VMEM((_GSZ,), jnp.int32),
            pltpu.VMEM((_GSZ,), jnp.float32),
            pltpu.VMEM((_GSZ,), jnp.float32),
            pltpu.VMEM((_P * _N_LEVELS * _F_PER,), jnp.float32),
            pltpu.SemaphoreType.DMA,
            pltpu.SemaphoreType.DMA,
        ],
    )
    return f(x, y, z, tab)


def kernel(inputs, table):
    x = inputs[:, 0]
    y = inputs[:, 1]
    z = inputs[:, 2]
    tab = (table.reshape(_N_LEVELS, _T // 128, 128, _F_PER)
           .transpose(0, 1, 3, 2)
           .reshape(_N_LEVELS * _T * _F_PER))
    out = _hash_grid(x, y, z, tab)
    return out.reshape(_N_POINTS, _N_LEVELS * _F_PER)
